# baseline (speedup 1.0000x reference)
import jax
import jax.numpy as jnp
from jax import lax
from jax.experimental import pallas as pl

_BS = 1024


def _fused_kernel(emb_ref, mask_ref, w_ref, out_ref):
    x = emb_ref[0].astype(jnp.bfloat16)
    wb = w_ref[...].astype(jnp.bfloat16)
    mm = lax.dot_general(wb, x, (((1,), (1,)), ((), ())),
                         preferred_element_type=jnp.float32)
    m = mask_ref[0] > 0
    out_ref[0] = jnp.where(m, mm, -jnp.inf)


def kernel(emb_sentences, att_sentences, W):
    B, S, D = emb_sentences.shape
    L = W.shape[0]
    mask = att_sentences[:, None, :].astype(jnp.float32)

    grid = (B, S // _BS)
    out_t = pl.pallas_call(
        _fused_kernel,
        grid=grid,
        in_specs=[
            pl.BlockSpec((1, _BS, D), lambda b, i: (b, i, 0)),
            pl.BlockSpec((1, 1, _BS), lambda b, i: (b, 0, i)),
            pl.BlockSpec((L, D), lambda b, i: (0, 0)),
        ],
        out_specs=pl.BlockSpec((1, L, _BS), lambda b, i: (b, 0, i)),
        out_shape=jax.ShapeDtypeStruct((B, L, S), jnp.float32),
    )(emb_sentences, mask, W)
    return jnp.swapaxes(out_t, 1, 2)


# speedup vs baseline: 1.1179x; 1.1179x over previous
import jax
import jax.numpy as jnp
from jax import lax
from jax.experimental import pallas as pl
import jax.experimental.pallas.tpu as pltpu

_BS = 2048
_NBUF = 4


def _fused_kernel(emb_hbm, mask_ref, w_ref, out_ref, *scratch):
    bufs = scratch[:_NBUF]
    sems = scratch[_NBUF:]
    B, L, S = out_ref.shape
    blocks_per_b = S // _BS
    nblocks = B * blocks_per_b
    wb = w_ref[...].astype(jnp.bfloat16)

    def copy(i):
        b, j = divmod(i, blocks_per_b)
        return pltpu.make_async_copy(
            emb_hbm.at[b, pl.ds(j * _BS, _BS), :], bufs[i % _NBUF], sems[i % _NBUF]
        )

    for i in range(min(_NBUF, nblocks)):
        copy(i).start()
    for i in range(nblocks):
        copy(i).wait()
        b, j = divmod(i, blocks_per_b)
        x = bufs[i % _NBUF][...].astype(jnp.bfloat16)
        mm = lax.dot_general(wb, x, (((1,), (1,)), ((), ())),
                             preferred_element_type=jnp.float32)
        m = mask_ref[b, :, pl.ds(j * _BS, _BS)] > 0
        out_ref[b, :, pl.ds(j * _BS, _BS)] = jnp.where(m, mm, -jnp.inf)
        if i + _NBUF < nblocks:
            copy(i + _NBUF).start()


def kernel(emb_sentences, att_sentences, W):
    B, S, D = emb_sentences.shape
    L = W.shape[0]
    mask = att_sentences[:, None, :].astype(jnp.float32)

    out_t = pl.pallas_call(
        _fused_kernel,
        in_specs=[
            pl.BlockSpec(memory_space=pl.ANY),
            pl.BlockSpec(memory_space=pltpu.MemorySpace.VMEM),
            pl.BlockSpec(memory_space=pltpu.MemorySpace.VMEM),
        ],
        out_specs=pl.BlockSpec(memory_space=pltpu.MemorySpace.VMEM),
        out_shape=jax.ShapeDtypeStruct((B, L, S), jnp.float32),
        scratch_shapes=[pltpu.VMEM((_BS, D), jnp.float32) for _ in range(_NBUF)]
        + [pltpu.SemaphoreType.DMA for _ in range(_NBUF)],
    )(emb_sentences, mask, W)
    return jnp.swapaxes(out_t, 1, 2)


# speedup vs baseline: 1.1803x; 1.0558x over previous
import jax
import jax.numpy as jnp
from jax import lax
from jax.experimental import pallas as pl
import jax.experimental.pallas.tpu as pltpu

_BS = 2048


def _fused_kernel(emb_ref, mask_ref, w_ref, out_ref):
    x = emb_ref[0].astype(jnp.bfloat16)
    wb = w_ref[...].astype(jnp.bfloat16)
    mm = lax.dot_general(wb, x, (((1,), (1,)), ((), ())),
                         preferred_element_type=jnp.float32)
    m = mask_ref[pl.ds(pl.program_id(0), 1), :]
    out_ref[0] = jnp.where(m, mm, -jnp.inf)


def kernel(emb_sentences, att_sentences, W):
    B, S, D = emb_sentences.shape
    L = W.shape[0]

    grid = (B, S // _BS)
    out_t = pl.pallas_call(
        _fused_kernel,
        grid=grid,
        in_specs=[
            pl.BlockSpec((1, _BS, D), lambda b, i: (b, i, 0)),
            pl.BlockSpec((B, _BS), lambda b, i: (0, i)),
            pl.BlockSpec((L, D), lambda b, i: (0, 0)),
        ],
        out_specs=pl.BlockSpec((1, L, _BS), lambda b, i: (b, 0, i)),
        out_shape=jax.ShapeDtypeStruct((B, L, S), jnp.float32),
        compiler_params=pltpu.CompilerParams(
            dimension_semantics=("parallel", "parallel")),
    )(emb_sentences, att_sentences, W)
    return jnp.swapaxes(out_t, 1, 2)
